# baseline (speedup 1.0000x reference)
import math

import jax
import jax.numpy as jnp
from jax.experimental import pallas as pl
from jax.experimental.pallas import tpu as pltpu

_G = 4
_NBUF = 4


def _phase1_body(nk, gb):
    def body(x0_ref, x1_ref, x2_ref, x3_ref, w_ref, xt_ref, wt_tail_ref,
             ts_ref, freqs_ref, embW_ref, embb_ref, wemb_ref, bin_ref,
             h_ref, ss_acc):
        k = pl.program_id(0)
        wb = w_ref[...]
        for g, xr in enumerate((x0_ref, x1_ref, x2_ref, x3_ref)):
            xb = xr[...]
            part = jnp.dot(xb.astype(jnp.bfloat16), wb,
                           preferred_element_type=jnp.float32)
            pss = jnp.sum(xb * xb, axis=1, keepdims=True)
            sl = slice(g * gb, (g + 1) * gb)

            @pl.when(k == 0)
            def _(sl=sl, part=part, pss=pss):
                h_ref[sl] = part
                ss_acc[sl] = pss

            @pl.when(k > 0)
            def _(sl=sl, part=part, pss=pss):
                h_ref[sl] = h_ref[sl] + part
                ss_acc[sl] = ss_acc[sl] + pss

        @pl.when(k == nk - 1)
        def _():
            xt = xt_ref[...]
            s = h_ref[...] + jnp.dot(xt, wt_tail_ref[...],
                                     preferred_element_type=jnp.float32)
            ss = ss_acc[...] + jnp.sum(xt * xt, axis=1, keepdims=True)
            t = ts_ref[...].astype(jnp.float32)
            temp = t * freqs_ref[...]
            te = jnp.concatenate([jnp.cos(temp), jnp.sin(temp)], axis=-1)
            emb = jnp.dot(te, embW_ref[...],
                          preferred_element_type=jnp.float32) + embb_ref[...]
            contrib = jnp.dot(emb, wemb_ref[...],
                              preferred_element_type=jnp.float32)
            norm = jnp.maximum(jnp.sqrt(ss), 1e-12)
            h_ref[...] = jnp.tanh(s / norm + contrib + bin_ref[...])

    return body


def _phase2_body(nr, tr):
    def body(h_ref, w_ref, b_ref, o_hbm, buf, sems):
        i = pl.program_id(0)
        slot = jax.lax.rem(i, _NBUF)

        @pl.when(i >= _NBUF)
        def _():
            pltpu.make_async_copy(
                buf.at[slot], o_hbm.at[pl.ds(i * tr, tr)], sems.at[slot],
            ).wait()

        o = jnp.dot(h_ref[...].astype(jnp.bfloat16), w_ref[...],
                    preferred_element_type=jnp.float32) + b_ref[...]
        buf[slot] = o
        pltpu.make_async_copy(
            buf.at[slot], o_hbm.at[pl.ds(i * tr, tr)], sems.at[slot],
        ).start()

        @pl.when(i == nr - 1)
        def _():
            for j in range(_NBUF):
                pltpu.make_async_copy(
                    buf.at[j], o_hbm.at[pl.ds(0, tr)], sems.at[j],
                ).wait()

    return body


def kernel(x, timesteps, emb_W, emb_b, W_in, b_in, W_out, b_out):
    B, ITEM = x.shape
    HID = W_out.shape[0]
    TD = emb_W.shape[0]
    half = TD // 2

    ALIGNED = (ITEM // 128) * 128
    TAIL = ITEM - ALIGNED
    bK = 1408
    NK = ALIGNED // bK
    GB = B // _G

    TR = 16
    NR = B // TR

    ts2 = timesteps.reshape(B, 1)
    freqs = jnp.exp(-(math.log(10000.0) / half)
                    * jnp.arange(half, dtype=jnp.float32)).reshape(1, half)
    W_main = jax.lax.slice(W_in, (0, 0), (ALIGNED, HID)).astype(jnp.bfloat16)
    W_tail = jax.lax.slice(W_in, (ALIGNED, 0), (ITEM, HID))
    x_tail = jax.lax.slice(x, (0, ALIGNED), (B, ITEM))
    W_emb = jax.lax.slice(W_in, (ITEM, 0), (ITEM + TD, HID))
    b_in2 = b_in.reshape(1, HID)
    emb_b2 = emb_b.reshape(1, TD)
    b_out2 = b_out.reshape(1, ITEM)
    W_out16 = W_out.astype(jnp.bfloat16)

    x_specs = [
        pl.BlockSpec((GB, bK), lambda k, g=g: (g, k)) for g in range(_G)
    ]

    h = pl.pallas_call(
        _phase1_body(NK, GB),
        grid=(NK,),
        in_specs=x_specs + [
            pl.BlockSpec((bK, HID), lambda k: (k, 0)),
            pl.BlockSpec((B, TAIL), lambda k: (0, 0)),
            pl.BlockSpec((TAIL, HID), lambda k: (0, 0)),
            pl.BlockSpec((B, 1), lambda k: (0, 0)),
            pl.BlockSpec((1, half), lambda k: (0, 0)),
            pl.BlockSpec((TD, TD), lambda k: (0, 0)),
            pl.BlockSpec((1, TD), lambda k: (0, 0)),
            pl.BlockSpec((TD, HID), lambda k: (0, 0)),
            pl.BlockSpec((1, HID), lambda k: (0, 0)),
        ],
        out_specs=pl.BlockSpec((B, HID), lambda k: (0, 0)),
        out_shape=jax.ShapeDtypeStruct((B, HID), jnp.float32),
        scratch_shapes=[pltpu.VMEM((B, 1), jnp.float32)],
        compiler_params=pltpu.CompilerParams(
            dimension_semantics=("arbitrary",)),
    )(x, x, x, x, W_main, x_tail, W_tail, ts2, freqs, emb_W, emb_b2, W_emb,
      b_in2)

    out = pl.pallas_call(
        _phase2_body(NR, TR),
        grid=(NR,),
        in_specs=[
            pl.BlockSpec((TR, HID), lambda r: (r, 0)),
            pl.BlockSpec((HID, ITEM), lambda r: (0, 0)),
            pl.BlockSpec((1, ITEM), lambda r: (0, 0)),
        ],
        out_specs=pl.BlockSpec(memory_space=pl.ANY),
        out_shape=jax.ShapeDtypeStruct((B, ITEM), jnp.float32),
        scratch_shapes=[
            pltpu.VMEM((_NBUF, TR, ITEM), jnp.float32),
            pltpu.SemaphoreType.DMA((_NBUF,)),
        ],
        compiler_params=pltpu.CompilerParams(
            dimension_semantics=("arbitrary",)),
    )(h, W_out16, b_out2)

    return out
